# Initial kernel scaffold; baseline (speedup 1.0000x reference)
import jax
import jax.numpy as jnp
from jax.experimental import pallas as pl


def kernel(user_id, item_id, user_table, item_table, ui_row, ui_col, ui_val, soc_row, soc_col, soc_val):
    raise NotImplementedError("write your pallas kernel here")


# speedup vs baseline: 1.9014x; 1.9014x over previous
import functools

import jax
import jax.numpy as jnp
from jax import lax
from jax.experimental import pallas as pl
from jax.experimental.pallas import tpu as pltpu
from jax.experimental.pallas import tpu_sc as plsc

N_NODES = 10000
N_PAD = 10240
D = 256
DH = 128
E = 160000
N_LAYERS = 3
SOCIAL_W = 0.3
BATCH = 4096

NC, NS = 2, 16
EPT = E // NS
K = 80
NCH = EPT // K
STRIPE = N_PAD // NS
ZR = 128

_f32 = jnp.float32
_i32 = jnp.int32


def _zero16():
    return jnp.zeros((16,), _f32)


def _propagation_kernel(usr2, itm2, ui_row, ui_col, ui_val,
                        soc_row, soc_col, soc_val,
                        u1, u2, u3, i1, i2, i3,
                        acc, idx_s, idx_d, valb, rows, zbuf, cbuf, sem):
    cid = lax.axis_index("c")
    sid = lax.axis_index("s")
    hoff = (cid * N_PAD).astype(_i32)

    zv = _zero16()

    def zrow(j, _):
        for t in range(DH // 16):
            zbuf[j, pl.ds(t * 16, 16)] = zv
        return 0

    lax.fori_loop(0, ZR, zrow, 0)

    def zero_acc():
        for j in range(STRIPE // ZR):
            r = sid * STRIPE + j * ZR
            pltpu.sync_copy(zbuf, acc.at[pl.ds(r, ZR)])

    def flush(dst):
        for j in range(STRIPE // ZR):
            r = sid * STRIPE + j * ZR
            pltpu.sync_copy(acc.at[pl.ds(r, ZR)], cbuf)
            pltpu.sync_copy(cbuf, dst.at[pl.ds(hoff + r, ZR)])

    def spmm(src_tab, src_idx_hbm, dst_idx_hbm, val_hbm, w):
        ebase = sid * EPT

        def chunk(c, _):
            off = pl.multiple_of(ebase + c * K, 8)
            pltpu.sync_copy(src_idx_hbm.at[pl.ds(off, K)], idx_s)
            pltpu.sync_copy(dst_idx_hbm.at[pl.ds(off, K)], idx_d)
            pltpu.sync_copy(val_hbm.at[pl.ds(off, K)], valb)
            for t in range(K // 16):
                sl = pl.ds(t * 16, 16)
                idx_s[sl] = idx_s[sl] + hoff
            pltpu.async_copy(src_tab.at[idx_s], rows, sem).wait()
            def edge4(i, _):
                for uu in range(4):
                    e = i * 4 + uu
                    bv = plsc.load_gather(valb, [jnp.full((16,), e, _i32)])
                    if w != 1.0:
                        bv = bv * w
                    for t in range(DH // 16):
                        sl = pl.ds(t * 16, 16)
                        rows[e, sl] = rows[e, sl] * bv
                return 0

            lax.fori_loop(0, K // 4, edge4, 0)
            pltpu.sync_copy(rows, acc.at[idx_d], add=True)
            return 0

        lax.fori_loop(0, NCH, chunk, 0)

    src_u, src_i = usr2, itm2
    layer_out = ((u1, i1), (u2, i2), (u3, i3))
    for k in range(N_LAYERS):
        u_out, i_out = layer_out[k]
        zero_acc()
        plsc.subcore_barrier()
        spmm(src_i, ui_col, ui_row, ui_val, 1.0 - SOCIAL_W)
        spmm(src_u, soc_col, soc_row, soc_val, SOCIAL_W)
        plsc.subcore_barrier()
        flush(u_out)
        zero_acc()
        plsc.subcore_barrier()
        spmm(src_u, ui_row, ui_col, ui_val, 1.0)
        plsc.subcore_barrier()
        flush(i_out)
        plsc.subcore_barrier()
        src_u, src_i = u_out, i_out


CB = BATCH // (NC * NS)
SB = 32


def _score_kernel(usr2, itm2, u1, u2, u3, i1, i2, i3, uid_hbm, iid_hbm,
                  scores, uidx, iidx, b0, b1, t1, t2, t3, sbuf, sem):
    cid = lax.axis_index("c")
    sid = lax.axis_index("s")
    wid = sid * NC + cid
    base = pl.multiple_of(wid * CB, 8)
    off10k = jnp.full((16,), N_PAD, _i32)

    def gather(tab, idx, dst):
        pltpu.async_copy(tab.at[idx], dst, sem).wait()

    def sum4(tab0, tabs, idx, dst):
        gather(tab0, idx, dst)
        gather(tabs[0], idx, t1)
        gather(tabs[1], idx, t2)
        gather(tabs[2], idx, t3)

        def addrow(r, _):
            for t in range(DH // 16):
                sl = pl.ds(t * 16, 16)
                dst[r, sl] = dst[r, sl] + t1[r, sl] + t2[r, sl] + t3[r, sl]
            return 0

        lax.fori_loop(0, SB, addrow, 0)

    def bump(idx):
        for t in range(SB // 16):
            sl = pl.ds(t * 16, 16)
            idx[sl] = idx[sl] + off10k

    lanes = jnp.arange(16, dtype=_i32)

    for s in range(CB // SB):
        off = base + s * SB
        pltpu.sync_copy(uid_hbm.at[pl.ds(off, SB)], uidx)
        pltpu.sync_copy(iid_hbm.at[pl.ds(off, SB)], iidx)

        for g in range(SB // 16):
            sbuf[pl.ds(s * SB + g * 16, 16)] = _zero16()

        for half in range(2):
            if half == 1:
                bump(uidx)
                bump(iidx)
            sum4(usr2, (u1, u2, u3), uidx, b0)
            sum4(itm2, (i1, i2, i3), iidx, b1)

            for g in range(SB // 16):
                rowv = lanes + (g * 16)

                def dbody(d, accv):
                    dv = jnp.full((16,), d, _i32)
                    uv = plsc.load_gather(b0, [rowv, dv])
                    iv = plsc.load_gather(b1, [rowv, dv])
                    return accv + uv * iv

                accv = lax.fori_loop(0, DH, dbody, _zero16())
                sl = pl.ds(s * SB + g * 16, 16)
                sbuf[sl] = sbuf[sl] + accv

    scale = 1.0 / float((N_LAYERS + 1) * (N_LAYERS + 1))
    for t in range(CB // 16):
        sl = pl.ds(t * 16, 16)
        sbuf[sl] = sbuf[sl] * scale
    pltpu.sync_copy(sbuf, scores.at[pl.ds(base, CB)])


_params = pltpu.CompilerParams(needs_layout_passes=False)

_mesh = plsc.VectorSubcoreMesh(core_axis_name="c", subcore_axis_name="s",
                               num_cores=NC, num_subcores=NS)

_tab = jax.ShapeDtypeStruct((NC * N_PAD, DH), _f32)

_propagate = functools.partial(
    pl.kernel, mesh=_mesh, compiler_params=_params,
    out_type=(_tab,) * 6,
    scratch_types=[
        pltpu.VMEM_SHARED((N_PAD, DH), _f32),
        pltpu.VMEM((K,), _i32),
        pltpu.VMEM((K,), _i32),
        pltpu.VMEM((K,), _f32),
        pltpu.VMEM((K, DH), _f32),
        pltpu.VMEM((ZR, DH), _f32),
        pltpu.VMEM((ZR, DH), _f32),
        pltpu.SemaphoreType.DMA,
    ],
)(_propagation_kernel)

_score = functools.partial(
    pl.kernel, mesh=_mesh, compiler_params=_params,
    out_type=jax.ShapeDtypeStruct((BATCH,), _f32),
    scratch_types=[
        pltpu.VMEM((SB,), _i32),
        pltpu.VMEM((SB,), _i32),
        pltpu.VMEM((SB, DH), _f32),
        pltpu.VMEM((SB, DH), _f32),
        pltpu.VMEM((SB, DH), _f32),
        pltpu.VMEM((SB, DH), _f32),
        pltpu.VMEM((SB, DH), _f32),
        pltpu.VMEM((CB,), _f32),
        pltpu.SemaphoreType.DMA,
    ],
)(_score_kernel)


def kernel(user_id, item_id, user_table, item_table,
           ui_row, ui_col, ui_val, soc_row, soc_col, soc_val):
    pad = ((0, 0), (0, N_PAD - N_NODES), (0, 0))
    usr2 = jnp.pad(user_table.reshape(N_NODES, NC, DH).transpose(1, 0, 2), pad)
    usr2 = usr2.reshape(NC * N_PAD, DH)
    itm2 = jnp.pad(item_table.reshape(N_NODES, NC, DH).transpose(1, 0, 2), pad)
    itm2 = itm2.reshape(NC * N_PAD, DH)

    ui_row = ui_row.astype(_i32)
    ui_col = ui_col.astype(_i32)
    soc_row = soc_row.astype(_i32)
    soc_col = soc_col.astype(_i32)

    u1, u2, u3, i1, i2, i3 = _propagate(
        usr2, itm2, ui_row, ui_col, ui_val, soc_row, soc_col, soc_val)
    return _score(usr2, itm2, u1, u2, u3, i1, i2, i3,
                  user_id.astype(_i32), item_id.astype(_i32))


# speedup vs baseline: 4.4992x; 2.3663x over previous
import functools

import jax
import jax.numpy as jnp
from jax import lax
from jax.experimental import pallas as pl
from jax.experimental.pallas import tpu as pltpu
from jax.experimental.pallas import tpu_sc as plsc

N_NODES = 10000
N_PAD = 10240
D = 256
DH = 128
E = 160000
N_LAYERS = 3
SOCIAL_W = 0.3
BATCH = 4096

NC, NS = 2, 16
EPT = E // NS
K = 40
NCH = EPT // K
STRIPE = N_PAD // NS
N_SLOTS = 2 * N_LAYERS

_f32 = jnp.float32
_i32 = jnp.int32


def _zero16():
    return jnp.zeros((16,), _f32)


def _propagation_kernel(usr2, itm2, es0, es1, es2, ed0, ed1, ed2,
                        ev0, ev1, ev2, tbuf,
                        acc, isb, idb, vlb,
                        r0, r1, r2, d0, d1, d2,
                        g0, g1, g2, s0, s1, s2):
    cid = lax.axis_index("c")
    sid = lax.axis_index("s")
    rowsb = (r0, r1, r2)
    dstb = (d0, d1, d2)
    gsem = (g0, g1, g2)
    ssem = (s0, s1, s2)
    zv = _zero16()
    ebase = pl.multiple_of(sid * EPT, 8)

    def zero_acc():
        def zrow(j, _):
            for t in range(DH // 16):
                r0[j, pl.ds(t * 16, 16)] = zv
            return 0

        lax.fori_loop(0, K, zrow, 0)
        for j in range(STRIPE // K):
            r = sid * STRIPE + j * K
            pltpu.sync_copy(r0, acc.at[pl.ds(r, K)])

    def flush(out_base):
        for j in range(STRIPE // K):
            r = sid * STRIPE + j * K
            pltpu.sync_copy(acc.at[pl.ds(r, K)],
                            tbuf.at[pl.ds(out_base + r, K)])

    def step_body(s, _):
        layer = s // 3
        phase = s - layer * 3
        g = jnp.where(phase == 1, 2, jnp.where(phase == 2, 1, 0))
        w = jnp.where(phase == 0, _f32(1.0 - SOCIAL_W),
                      jnp.where(phase == 1, _f32(SOCIAL_W), _f32(1.0)))
        use_itm2 = s == 0
        use_usr2 = (s == 1) | (s == 2)
        use_tbuf = s >= 3
        src_slot = jnp.maximum(
            jnp.where(phase == 0, 2 * layer - 1, 2 * layer - 2), 0)
        row_base = jnp.where(
            use_tbuf, (src_slot * 2 + cid) * N_PAD, cid * N_PAD).astype(_i32)
        out_slot = jnp.where(phase == 2, 2 * layer + 1, 2 * layer)
        out_base = (out_slot * 2 + cid) * N_PAD

        @pl.when(phase != 1)
        def _():
            zero_acc()

        plsc.subcore_barrier()

        for gg, (es, ed, ev) in enumerate(
                ((es0, ed0, ev0), (es1, ed1, ev1), (es2, ed2, ev2))):
            @pl.when(g == gg)
            def _():
                pltpu.sync_copy(es.at[pl.ds(ebase, EPT)], isb)
                pltpu.sync_copy(ed.at[pl.ds(ebase, EPT)], idb)
                pltpu.sync_copy(ev.at[pl.ds(ebase, EPT)], vlb)

        def adj(t, _):
            for u in range(5):
                sl = pl.ds((t * 5 + u) * 16, 16)
                isb[sl] = isb[sl] + row_base
            return 0

        lax.fori_loop(0, EPT // 80, adj, 0)

        def gidx(c):
            return isb.at[pl.ds(pl.multiple_of(c * K, 8), K)]

        def gstart(c, b):
            @pl.when(use_itm2)
            def _():
                pltpu.async_copy(itm2.at[gidx(c)], rowsb[b], gsem[b])

            @pl.when(use_usr2)
            def _():
                pltpu.async_copy(usr2.at[gidx(c)], rowsb[b], gsem[b])

            @pl.when(use_tbuf)
            def _():
                pltpu.async_copy(tbuf.at[gidx(c)], rowsb[b], gsem[b])

        def gwait(c, b):
            @pl.when(use_itm2)
            def _():
                pltpu.make_async_copy(itm2.at[gidx(c)], rowsb[b],
                                      gsem[b]).wait()

            @pl.when(use_usr2)
            def _():
                pltpu.make_async_copy(usr2.at[gidx(c)], rowsb[b],
                                      gsem[b]).wait()

            @pl.when(use_tbuf)
            def _():
                pltpu.make_async_copy(tbuf.at[gidx(c)], rowsb[b],
                                      gsem[b]).wait()

        def fill_dst(c, b):
            for o in (0, 16, 24):
                dstb[b][pl.ds(o, 16)] = idb[pl.ds(c * K + o, 16)]

        def sstart(b):
            pltpu.async_copy(rowsb[b], acc.at[dstb[b]], ssem[b], add=True)

        def swait(b):
            pltpu.make_async_copy(rowsb[b], acc.at[dstb[b]], ssem[b]).wait()

        def scale(c, b):
            rb = rowsb[b]

            def edge4(i, _):
                for uu in range(4):
                    e = i * 4 + uu
                    bv = plsc.load_gather(
                        vlb, [jnp.full((16,), c * K + e, _i32)]) * w
                    for t in range(DH // 16):
                        sl = pl.ds(t * 16, 16)
                        rb[e, sl] = rb[e, sl] * bv
                return 0

            lax.fori_loop(0, K // 4, edge4, 0)

        def process(c, b, prefetch):
            gwait(c, b)
            fill_dst(c, b)
            scale(c, b)
            sstart(b)
            if prefetch:
                b2 = (b + 2) % 3
                if isinstance(c, int):
                    if c >= 1:
                        swait(b2)
                else:
                    @pl.when(c >= 1)
                    def _():
                        swait(b2)

                gstart(c + 2, b2)

        gstart(0, 0)
        gstart(1, 1)

        def triple(i, _):
            c0 = i * 3
            process(c0, 0, True)
            process(c0 + 1, 1, True)
            process(c0 + 2, 2, True)
            return 0

        lax.fori_loop(0, (NCH - 4) // 3, triple, 0)
        process(NCH - 4, (NCH - 4) % 3, True)
        process(NCH - 3, (NCH - 3) % 3, True)
        process(NCH - 2, (NCH - 2) % 3, False)
        process(NCH - 1, (NCH - 1) % 3, False)
        swait((NCH - 3) % 3)
        swait((NCH - 2) % 3)
        swait((NCH - 1) % 3)

        plsc.subcore_barrier()

        @pl.when(phase != 0)
        def _():
            flush(out_base)

        return 0

    lax.fori_loop(0, 3 * N_LAYERS, step_body, 0)


CB = BATCH // (NC * NS)
SB = 32


def _score_kernel(usr2, itm2, u1, u2, u3, i1, i2, i3, uid_hbm, iid_hbm,
                  scores, uidx, iidx, b0, b1, t1, t2, t3, sbuf, sem):
    cid = lax.axis_index("c")
    sid = lax.axis_index("s")
    wid = sid * NC + cid
    base = pl.multiple_of(wid * CB, 8)
    off10k = jnp.full((16,), N_PAD, _i32)

    def gather(tab, idx, dst):
        pltpu.async_copy(tab.at[idx], dst, sem).wait()

    def sum4(tab0, tabs, idx, dst):
        gather(tab0, idx, dst)
        gather(tabs[0], idx, t1)
        gather(tabs[1], idx, t2)
        gather(tabs[2], idx, t3)

        def addrow(r, _):
            for t in range(DH // 16):
                sl = pl.ds(t * 16, 16)
                dst[r, sl] = dst[r, sl] + t1[r, sl] + t2[r, sl] + t3[r, sl]
            return 0

        lax.fori_loop(0, SB, addrow, 0)

    def bump(idx):
        for t in range(SB // 16):
            sl = pl.ds(t * 16, 16)
            idx[sl] = idx[sl] + off10k

    lanes = jnp.arange(16, dtype=_i32)

    for s in range(CB // SB):
        off = base + s * SB
        pltpu.sync_copy(uid_hbm.at[pl.ds(off, SB)], uidx)
        pltpu.sync_copy(iid_hbm.at[pl.ds(off, SB)], iidx)

        for g in range(SB // 16):
            sbuf[pl.ds(s * SB + g * 16, 16)] = _zero16()

        for half in range(2):
            if half == 1:
                bump(uidx)
                bump(iidx)
            sum4(usr2, (u1, u2, u3), uidx, b0)
            sum4(itm2, (i1, i2, i3), iidx, b1)

            for g in range(SB // 16):
                rowv = lanes + (g * 16)

                def dbody(d, accv):
                    dv = jnp.full((16,), d, _i32)
                    uv = plsc.load_gather(b0, [rowv, dv])
                    iv = plsc.load_gather(b1, [rowv, dv])
                    return accv + uv * iv

                accv = lax.fori_loop(0, DH, dbody, _zero16())
                sl = pl.ds(s * SB + g * 16, 16)
                sbuf[sl] = sbuf[sl] + accv

    scale = 1.0 / float((N_LAYERS + 1) * (N_LAYERS + 1))
    for t in range(CB // 16):
        sl = pl.ds(t * 16, 16)
        sbuf[sl] = sbuf[sl] * scale
    pltpu.sync_copy(sbuf, scores.at[pl.ds(base, CB)])


_params = pltpu.CompilerParams(needs_layout_passes=False)

_mesh = plsc.VectorSubcoreMesh(core_axis_name="c", subcore_axis_name="s",
                               num_cores=NC, num_subcores=NS)

_propagate = functools.partial(
    pl.kernel, mesh=_mesh, compiler_params=_params,
    out_type=jax.ShapeDtypeStruct((N_SLOTS * NC * N_PAD, DH), _f32),
    scratch_types=[
        pltpu.VMEM_SHARED((N_PAD, DH), _f32),
        pltpu.VMEM((EPT,), _i32),
        pltpu.VMEM((EPT,), _i32),
        pltpu.VMEM((EPT,), _f32),
        pltpu.VMEM((K, DH), _f32),
        pltpu.VMEM((K, DH), _f32),
        pltpu.VMEM((K, DH), _f32),
        pltpu.VMEM((K,), _i32),
        pltpu.VMEM((K,), _i32),
        pltpu.VMEM((K,), _i32),
        pltpu.SemaphoreType.DMA,
        pltpu.SemaphoreType.DMA,
        pltpu.SemaphoreType.DMA,
        pltpu.SemaphoreType.DMA,
        pltpu.SemaphoreType.DMA,
        pltpu.SemaphoreType.DMA,
    ],
)(_propagation_kernel)

_score = functools.partial(
    pl.kernel, mesh=_mesh, compiler_params=_params,
    out_type=jax.ShapeDtypeStruct((BATCH,), _f32),
    scratch_types=[
        pltpu.VMEM((SB,), _i32),
        pltpu.VMEM((SB,), _i32),
        pltpu.VMEM((SB, DH), _f32),
        pltpu.VMEM((SB, DH), _f32),
        pltpu.VMEM((SB, DH), _f32),
        pltpu.VMEM((SB, DH), _f32),
        pltpu.VMEM((SB, DH), _f32),
        pltpu.VMEM((CB,), _f32),
        pltpu.SemaphoreType.DMA,
    ],
)(_score_kernel)


def kernel(user_id, item_id, user_table, item_table,
           ui_row, ui_col, ui_val, soc_row, soc_col, soc_val):
    pad = ((0, 0), (0, N_PAD - N_NODES), (0, 0))
    usr2 = jnp.pad(user_table.reshape(N_NODES, NC, DH).transpose(1, 0, 2),
                   pad)
    usr2 = usr2.reshape(NC * N_PAD, DH)
    itm2 = jnp.pad(item_table.reshape(N_NODES, NC, DH).transpose(1, 0, 2),
                   pad)
    itm2 = itm2.reshape(NC * N_PAD, DH)

    ui_row = ui_row.astype(_i32)
    ui_col = ui_col.astype(_i32)
    soc_row = soc_row.astype(_i32)
    soc_col = soc_col.astype(_i32)

    tbuf = _propagate(usr2, itm2,
                      ui_col, ui_row, soc_col,
                      ui_row, ui_col, soc_row,
                      ui_val, ui_val, soc_val)
    t6 = tbuf.reshape(N_SLOTS, NC * N_PAD, DH)
    return _score(usr2, itm2, t6[0], t6[2], t6[4], t6[1], t6[3], t6[5],
                  user_id.astype(_i32), item_id.astype(_i32))


# speedup vs baseline: 5.7196x; 1.2713x over previous
import functools

import jax
import jax.numpy as jnp
from jax import lax
from jax.experimental import pallas as pl
from jax.experimental.pallas import tpu as pltpu
from jax.experimental.pallas import tpu_sc as plsc

N_NODES = 10000
N_PAD = 10240
D = 256
DH = 128
E = 160000
N_LAYERS = 3
SOCIAL_W = 0.3
BATCH = 4096

NC, NS = 2, 16
EPT = E // NS
K = 80
NCH = EPT // K
STRIPE = N_PAD // NS
N_SLOTS = 2 * N_LAYERS

_f32 = jnp.float32
_i32 = jnp.int32


def _zero16():
    return jnp.zeros((16,), _f32)


def _propagation_kernel(usr2, itm2, es0, es1, es2, ed0, ed1, ed2,
                        ev0, ev1, ev2, tbuf,
                        acc, isb,
                        r0, r1, r2, d0, d1, d2, v0, v1, v2,
                        g0, g1, g2, s0, s1, s2, e0, e1, e2):
    cid = lax.axis_index("c")
    sid = lax.axis_index("s")
    rowsb = (r0, r1, r2)
    dstb = (d0, d1, d2)
    valb = (v0, v1, v2)
    gsem = (g0, g1, g2)
    ssem = (s0, s1, s2)
    esem = (e0, e1, e2)
    zv = _zero16()
    ebase = pl.multiple_of(sid * EPT, 8)

    def zero_acc():
        def zrow(j, _):
            for t in range(DH // 16):
                r0[j, pl.ds(t * 16, 16)] = zv
            return 0

        lax.fori_loop(0, K, zrow, 0)
        for j in range(STRIPE // K):
            r = sid * STRIPE + j * K
            pltpu.sync_copy(r0, acc.at[pl.ds(r, K)])

    def flush(out_base):
        for j in range(STRIPE // K):
            r = sid * STRIPE + j * K
            pltpu.sync_copy(acc.at[pl.ds(r, K)],
                            tbuf.at[pl.ds(out_base + r, K)])

    def step_body(s, _):
        layer = s // 3
        phase = s - layer * 3
        g = jnp.where(phase == 1, 2, jnp.where(phase == 2, 1, 0))
        w = jnp.where(phase == 0, _f32(1.0 - SOCIAL_W),
                      jnp.where(phase == 1, _f32(SOCIAL_W), _f32(1.0)))
        use_itm2 = s == 0
        use_usr2 = (s == 1) | (s == 2)
        use_tbuf = s >= 3
        src_slot = jnp.maximum(
            jnp.where(phase == 0, 2 * layer - 1, 2 * layer - 2), 0)
        row_base = jnp.where(
            use_tbuf, (src_slot * 2 + cid) * N_PAD, cid * N_PAD).astype(_i32)
        out_slot = jnp.where(phase == 2, 2 * layer + 1, 2 * layer)
        out_base = (out_slot * 2 + cid) * N_PAD

        @pl.when(phase != 1)
        def _():
            zero_acc()

        plsc.subcore_barrier()

        for gg, es in enumerate((es0, es1, es2)):
            @pl.when(g == gg)
            def _():
                pltpu.sync_copy(es.at[pl.ds(ebase, EPT)], isb)

        def adj(t, _):
            for u in range(5):
                sl = pl.ds((t * 5 + u) * 16, 16)
                isb[sl] = isb[sl] + row_base
            return 0

        lax.fori_loop(0, EPT // 80, adj, 0)

        def gidx(c):
            return isb.at[pl.ds(pl.multiple_of(c * K, 8), K)]

        def gstart(c, b):
            @pl.when(use_itm2)
            def _():
                pltpu.async_copy(itm2.at[gidx(c)], rowsb[b], gsem[b])

            @pl.when(use_usr2)
            def _():
                pltpu.async_copy(usr2.at[gidx(c)], rowsb[b], gsem[b])

            @pl.when(use_tbuf)
            def _():
                pltpu.async_copy(tbuf.at[gidx(c)], rowsb[b], gsem[b])

        def gwait(c, b):
            @pl.when(use_itm2)
            def _():
                pltpu.make_async_copy(itm2.at[gidx(c)], rowsb[b],
                                      gsem[b]).wait()

            @pl.when(use_usr2)
            def _():
                pltpu.make_async_copy(usr2.at[gidx(c)], rowsb[b],
                                      gsem[b]).wait()

            @pl.when(use_tbuf)
            def _():
                pltpu.make_async_copy(tbuf.at[gidx(c)], rowsb[b],
                                      gsem[b]).wait()

        def side_start(c, b):
            eoff = pl.multiple_of(ebase + c * K, 8)
            for gg, (ed, ev) in enumerate(
                    ((ed0, ev0), (ed1, ev1), (ed2, ev2))):
                @pl.when(g == gg)
                def _():
                    pltpu.async_copy(ed.at[pl.ds(eoff, K)], dstb[b], esem[b])
                    pltpu.async_copy(ev.at[pl.ds(eoff, K)], valb[b], esem[b])

        def side_wait(b):
            pltpu.make_async_copy(ed0.at[pl.ds(0, K)], dstb[b],
                                  esem[b]).wait()
            pltpu.make_async_copy(ev0.at[pl.ds(0, K)], valb[b],
                                  esem[b]).wait()

        def sstart(b):
            pltpu.async_copy(rowsb[b], acc.at[dstb[b]], ssem[b], add=True)

        def swait(b):
            pltpu.make_async_copy(rowsb[b], acc.at[dstb[b]], ssem[b]).wait()

        def scale(b):
            rb = rowsb[b]
            vb = valb[b]

            def edge4(i, _):
                for uu in range(4):
                    e = i * 4 + uu
                    bv = plsc.load_gather(
                        vb, [jnp.full((16,), e, _i32)]) * w
                    for t in range(DH // 16):
                        sl = pl.ds(t * 16, 16)
                        rb[e, sl] = rb[e, sl] * bv
                return 0

            lax.fori_loop(0, K // 4, edge4, 0)

        def process(c, b, prefetch):
            gwait(c, b)
            side_wait(b)
            scale(b)
            sstart(b)
            if prefetch:
                b2 = (b + 2) % 3
                if isinstance(c, int):
                    if c >= 1:
                        swait(b2)
                else:
                    @pl.when(c >= 1)
                    def _():
                        swait(b2)

                side_start(c + 2, b2)
                gstart(c + 2, b2)

        side_start(0, 0)
        gstart(0, 0)
        side_start(1, 1)
        gstart(1, 1)

        def triple(i, _):
            c0 = i * 3
            process(c0, 0, True)
            process(c0 + 1, 1, True)
            process(c0 + 2, 2, True)
            return 0

        lax.fori_loop(0, (NCH - 2) // 3, triple, 0)
        process(NCH - 2, (NCH - 2) % 3, False)
        process(NCH - 1, (NCH - 1) % 3, False)
        swait((NCH - 3) % 3)
        swait((NCH - 2) % 3)
        swait((NCH - 1) % 3)

        plsc.subcore_barrier()

        @pl.when(phase != 0)
        def _():
            flush(out_base)

        return 0

    lax.fori_loop(0, 3 * N_LAYERS, step_body, 0)


CB = BATCH // (NC * NS)
SB = 32


def _score_kernel(usr2, itm2, u1, u2, u3, i1, i2, i3, uid_hbm, iid_hbm,
                  scores, uidx, iidx, b0, b1, t1, t2, t3, sbuf, sem):
    cid = lax.axis_index("c")
    sid = lax.axis_index("s")
    wid = sid * NC + cid
    base = pl.multiple_of(wid * CB, 8)
    off10k = jnp.full((16,), N_PAD, _i32)

    def gather(tab, idx, dst):
        pltpu.async_copy(tab.at[idx], dst, sem).wait()

    def sum4(tab0, tabs, idx, dst):
        gather(tab0, idx, dst)
        gather(tabs[0], idx, t1)
        gather(tabs[1], idx, t2)
        gather(tabs[2], idx, t3)

        def addrow(r, _):
            for t in range(DH // 16):
                sl = pl.ds(t * 16, 16)
                dst[r, sl] = dst[r, sl] + t1[r, sl] + t2[r, sl] + t3[r, sl]
            return 0

        lax.fori_loop(0, SB, addrow, 0)

    def bump(idx):
        for t in range(SB // 16):
            sl = pl.ds(t * 16, 16)
            idx[sl] = idx[sl] + off10k

    lanes = jnp.arange(16, dtype=_i32)

    for s in range(CB // SB):
        off = base + s * SB
        pltpu.sync_copy(uid_hbm.at[pl.ds(off, SB)], uidx)
        pltpu.sync_copy(iid_hbm.at[pl.ds(off, SB)], iidx)

        for g in range(SB // 16):
            sbuf[pl.ds(s * SB + g * 16, 16)] = _zero16()

        for half in range(2):
            if half == 1:
                bump(uidx)
                bump(iidx)
            sum4(usr2, (u1, u2, u3), uidx, b0)
            sum4(itm2, (i1, i2, i3), iidx, b1)

            for g in range(SB // 16):
                rowv = lanes + (g * 16)

                def dbody(d, accv):
                    dv = jnp.full((16,), d, _i32)
                    uv = plsc.load_gather(b0, [rowv, dv])
                    iv = plsc.load_gather(b1, [rowv, dv])
                    return accv + uv * iv

                accv = lax.fori_loop(0, DH, dbody, _zero16())
                sl = pl.ds(s * SB + g * 16, 16)
                sbuf[sl] = sbuf[sl] + accv

    scale = 1.0 / float((N_LAYERS + 1) * (N_LAYERS + 1))
    for t in range(CB // 16):
        sl = pl.ds(t * 16, 16)
        sbuf[sl] = sbuf[sl] * scale
    pltpu.sync_copy(sbuf, scores.at[pl.ds(base, CB)])


_params = pltpu.CompilerParams(needs_layout_passes=False)

_mesh = plsc.VectorSubcoreMesh(core_axis_name="c", subcore_axis_name="s",
                               num_cores=NC, num_subcores=NS)

_propagate = functools.partial(
    pl.kernel, mesh=_mesh, compiler_params=_params,
    out_type=jax.ShapeDtypeStruct((N_SLOTS * NC * N_PAD, DH), _f32),
    scratch_types=[
        pltpu.VMEM_SHARED((N_PAD, DH), _f32),
        pltpu.VMEM((EPT,), _i32),
        pltpu.VMEM((K, DH), _f32),
        pltpu.VMEM((K, DH), _f32),
        pltpu.VMEM((K, DH), _f32),
        pltpu.VMEM((K,), _i32),
        pltpu.VMEM((K,), _i32),
        pltpu.VMEM((K,), _i32),
        pltpu.VMEM((K,), _f32),
        pltpu.VMEM((K,), _f32),
        pltpu.VMEM((K,), _f32),
        pltpu.SemaphoreType.DMA,
        pltpu.SemaphoreType.DMA,
        pltpu.SemaphoreType.DMA,
        pltpu.SemaphoreType.DMA,
        pltpu.SemaphoreType.DMA,
        pltpu.SemaphoreType.DMA,
        pltpu.SemaphoreType.DMA,
        pltpu.SemaphoreType.DMA,
        pltpu.SemaphoreType.DMA,
    ],
)(_propagation_kernel)

_score = functools.partial(
    pl.kernel, mesh=_mesh, compiler_params=_params,
    out_type=jax.ShapeDtypeStruct((BATCH,), _f32),
    scratch_types=[
        pltpu.VMEM((SB,), _i32),
        pltpu.VMEM((SB,), _i32),
        pltpu.VMEM((SB, DH), _f32),
        pltpu.VMEM((SB, DH), _f32),
        pltpu.VMEM((SB, DH), _f32),
        pltpu.VMEM((SB, DH), _f32),
        pltpu.VMEM((SB, DH), _f32),
        pltpu.VMEM((CB,), _f32),
        pltpu.SemaphoreType.DMA,
    ],
)(_score_kernel)


def kernel(user_id, item_id, user_table, item_table,
           ui_row, ui_col, ui_val, soc_row, soc_col, soc_val):
    pad = ((0, 0), (0, N_PAD - N_NODES), (0, 0))
    usr2 = jnp.pad(user_table.reshape(N_NODES, NC, DH).transpose(1, 0, 2),
                   pad)
    usr2 = usr2.reshape(NC * N_PAD, DH)
    itm2 = jnp.pad(item_table.reshape(N_NODES, NC, DH).transpose(1, 0, 2),
                   pad)
    itm2 = itm2.reshape(NC * N_PAD, DH)

    ui_row = ui_row.astype(_i32)
    ui_col = ui_col.astype(_i32)
    soc_row = soc_row.astype(_i32)
    soc_col = soc_col.astype(_i32)

    tbuf = _propagate(usr2, itm2,
                      ui_col, ui_row, soc_col,
                      ui_row, ui_col, soc_row,
                      ui_val, ui_val, soc_val)
    t6 = tbuf.reshape(N_SLOTS, NC * N_PAD, DH)
    return _score(usr2, itm2, t6[0], t6[2], t6[4], t6[1], t6[3], t6[5],
                  user_id.astype(_i32), item_id.astype(_i32))


# speedup vs baseline: 8.0389x; 1.4055x over previous
import functools

import jax
import jax.numpy as jnp
from jax import lax
from jax.experimental import pallas as pl
from jax.experimental.pallas import tpu as pltpu
from jax.experimental.pallas import tpu_sc as plsc

N_NODES = 10000
N_PAD = 10240
D = 256
DH = 128
E = 160000
N_LAYERS = 3
SOCIAL_W = 0.3
BATCH = 4096

NC, NS = 2, 16
EPT = E // NS
K = 80
NCH = EPT // K
STRIPE = N_PAD // NS
N_SLOTS = 2 * N_LAYERS

_f32 = jnp.float32
_i32 = jnp.int32


def _zero16():
    return jnp.zeros((16,), _f32)


def _propagation_kernel(usr2, itm2, es0, es1, es2, ed0, ed1, ed2,
                        ev0, ev1, ev2, tbuf,
                        acc, isb,
                        r0, r1, r2, d0, d1, d2, v0, v1, v2,
                        g0, g1, g2, s0, s1, s2, e0, e1, e2):
    cid = lax.axis_index("c")
    sid = lax.axis_index("s")
    rowsb = (r0, r1, r2)
    dstb = (d0, d1, d2)
    valb = (v0, v1, v2)
    gsem = (g0, g1, g2)
    ssem = (s0, s1, s2)
    esem = (e0, e1, e2)
    zv = _zero16()
    ebase = pl.multiple_of(sid * EPT, 8)

    def zero_acc():
        def zrow(j, _):
            for t in range(DH // 16):
                r0[j, pl.ds(t * 16, 16)] = zv
            return 0

        lax.fori_loop(0, K, zrow, 0)
        for j in range(STRIPE // K):
            r = sid * STRIPE + j * K
            pltpu.async_copy(r0, acc.at[pl.ds(r, K)], s0)
        for j in range(STRIPE // K):
            r = sid * STRIPE + j * K
            pltpu.make_async_copy(r0, acc.at[pl.ds(r, K)], s0).wait()

    def flush(out_base):
        for j in range(STRIPE // K):
            r = sid * STRIPE + j * K
            pltpu.async_copy(acc.at[pl.ds(r, K)],
                             tbuf.at[pl.ds(out_base + r, K)], s1)
        for j in range(STRIPE // K):
            r = sid * STRIPE + j * K
            pltpu.make_async_copy(acc.at[pl.ds(r, K)],
                                  tbuf.at[pl.ds(out_base + r, K)], s1).wait()

    def step_body(s, _):
        layer = s // 3
        phase = s - layer * 3
        g = jnp.where(phase == 1, 2, jnp.where(phase == 2, 1, 0))
        use_itm2 = s == 0
        use_usr2 = (s == 1) | (s == 2)
        use_tbuf = s >= 3
        src_slot = jnp.maximum(
            jnp.where(phase == 0, 2 * layer - 1, 2 * layer - 2), 0)
        row_base = jnp.where(
            use_tbuf, (src_slot * 2 + cid) * N_PAD, cid * N_PAD).astype(_i32)
        out_slot = jnp.where(phase == 2, 2 * layer + 1, 2 * layer)
        out_base = (out_slot * 2 + cid) * N_PAD

        @pl.when(phase != 1)
        def _():
            zero_acc()

        plsc.subcore_barrier()

        for gg, es in enumerate((es0, es1, es2)):
            @pl.when(g == gg)
            def _():
                pltpu.sync_copy(es.at[pl.ds(ebase, EPT)], isb)

        rb8 = pl.multiple_of(row_base, 8)

        def gidx(c):
            return isb.at[pl.ds(pl.multiple_of(c * K, 8), K)]

        def gstart(c, b):
            @pl.when(use_itm2)
            def _():
                pltpu.async_copy(
                    itm2.at[pl.ds(rb8, N_PAD)].at[gidx(c)],
                    rowsb[b], gsem[b])

            @pl.when(use_usr2)
            def _():
                pltpu.async_copy(
                    usr2.at[pl.ds(rb8, N_PAD)].at[gidx(c)],
                    rowsb[b], gsem[b])

            @pl.when(use_tbuf)
            def _():
                pltpu.async_copy(
                    tbuf.at[pl.ds(rb8, N_PAD)].at[gidx(c)],
                    rowsb[b], gsem[b])

        def gwait(c, b):
            pltpu.make_async_copy(
                itm2.at[pl.ds(0, N_PAD)].at[gidx(c)], rowsb[b],
                gsem[b]).wait()

        def side_start(c, b):
            eoff = pl.multiple_of(ebase + c * K, 8)
            for gg, (ed, ev) in enumerate(
                    ((ed0, ev0), (ed1, ev1), (ed2, ev2))):
                @pl.when(g == gg)
                def _():
                    pltpu.async_copy(ed.at[pl.ds(eoff, K)], dstb[b], esem[b])
                    pltpu.async_copy(ev.at[pl.ds(eoff, K)], valb[b], esem[b])

        def side_wait(b):
            pltpu.make_async_copy(ed0.at[pl.ds(0, K)], dstb[b],
                                  esem[b]).wait()
            pltpu.make_async_copy(ev0.at[pl.ds(0, K)], valb[b],
                                  esem[b]).wait()

        def sstart(b):
            pltpu.async_copy(rowsb[b], acc.at[dstb[b]], ssem[b], add=True)

        def swait(b):
            pltpu.make_async_copy(rowsb[b], acc.at[dstb[b]], ssem[b]).wait()

        def scale(b):
            rb = rowsb[b]
            vb = valb[b]

            @functools.partial(plsc.parallel_loop, 0, K, unroll=8)
            def _(e):
                bv = plsc.load_gather(vb, [jnp.full((16,), e, _i32)])
                for t in range(DH // 16):
                    sl = pl.ds(t * 16, 16)
                    rb[e, sl] = rb[e, sl] * bv

        def process(c, b, prefetch):
            gwait(c, b)
            side_wait(b)
            scale(b)
            sstart(b)
            if prefetch:
                b2 = (b + 2) % 3
                if isinstance(c, int):
                    if c >= 1:
                        swait(b2)
                else:
                    @pl.when(c >= 1)
                    def _():
                        swait(b2)

                side_start(c + 2, b2)
                gstart(c + 2, b2)

        side_start(0, 0)
        gstart(0, 0)
        side_start(1, 1)
        gstart(1, 1)

        def triple(i, _):
            c0 = i * 3
            process(c0, 0, True)
            process(c0 + 1, 1, True)
            process(c0 + 2, 2, True)
            return 0

        lax.fori_loop(0, (NCH - 2) // 3, triple, 0)
        process(NCH - 2, (NCH - 2) % 3, False)
        process(NCH - 1, (NCH - 1) % 3, False)
        swait((NCH - 3) % 3)
        swait((NCH - 2) % 3)
        swait((NCH - 1) % 3)

        plsc.subcore_barrier()

        @pl.when(phase != 0)
        def _():
            flush(out_base)

        return 0

    lax.fori_loop(0, 3 * N_LAYERS, step_body, 0)


CB = BATCH // (NC * NS)
SB = 32


def _score_kernel(usr2, itm2, u1, u2, u3, i1, i2, i3, uid_hbm, iid_hbm,
                  scores, uidx, iidx, b0, b1, t1, t2, t3, sbuf, sem):
    cid = lax.axis_index("c")
    sid = lax.axis_index("s")
    wid = sid * NC + cid
    base = pl.multiple_of(wid * CB, 8)
    off10k = jnp.full((16,), N_PAD, _i32)

    def gather(tab, idx, dst):
        pltpu.async_copy(tab.at[idx], dst, sem).wait()

    def sum4(tab0, tabs, idx, dst):
        gather(tab0, idx, dst)
        gather(tabs[0], idx, t1)
        gather(tabs[1], idx, t2)
        gather(tabs[2], idx, t3)

        def addrow(r, _):
            for t in range(DH // 16):
                sl = pl.ds(t * 16, 16)
                dst[r, sl] = dst[r, sl] + t1[r, sl] + t2[r, sl] + t3[r, sl]
            return 0

        lax.fori_loop(0, SB, addrow, 0)

    def bump(idx):
        for t in range(SB // 16):
            sl = pl.ds(t * 16, 16)
            idx[sl] = idx[sl] + off10k

    lanes = jnp.arange(16, dtype=_i32)

    for s in range(CB // SB):
        off = base + s * SB
        pltpu.sync_copy(uid_hbm.at[pl.ds(off, SB)], uidx)
        pltpu.sync_copy(iid_hbm.at[pl.ds(off, SB)], iidx)

        for g in range(SB // 16):
            sbuf[pl.ds(s * SB + g * 16, 16)] = _zero16()

        for half in range(2):
            if half == 1:
                bump(uidx)
                bump(iidx)
            sum4(usr2, (u1, u2, u3), uidx, b0)
            sum4(itm2, (i1, i2, i3), iidx, b1)

            for g in range(SB // 16):
                rowv = lanes + (g * 16)

                def dbody(d, accv):
                    dv = jnp.full((16,), d, _i32)
                    uv = plsc.load_gather(b0, [rowv, dv])
                    iv = plsc.load_gather(b1, [rowv, dv])
                    return accv + uv * iv

                accv = lax.fori_loop(0, DH, dbody, _zero16())
                sl = pl.ds(s * SB + g * 16, 16)
                sbuf[sl] = sbuf[sl] + accv

    scale = 1.0 / float((N_LAYERS + 1) * (N_LAYERS + 1))
    for t in range(CB // 16):
        sl = pl.ds(t * 16, 16)
        sbuf[sl] = sbuf[sl] * scale
    pltpu.sync_copy(sbuf, scores.at[pl.ds(base, CB)])


_params = pltpu.CompilerParams(needs_layout_passes=False)

_mesh = plsc.VectorSubcoreMesh(core_axis_name="c", subcore_axis_name="s",
                               num_cores=NC, num_subcores=NS)

_propagate = functools.partial(
    pl.kernel, mesh=_mesh, compiler_params=_params,
    out_type=jax.ShapeDtypeStruct((N_SLOTS * NC * N_PAD, DH), _f32),
    scratch_types=[
        pltpu.VMEM_SHARED((N_PAD, DH), _f32),
        pltpu.VMEM((EPT,), _i32),
        pltpu.VMEM((K, DH), _f32),
        pltpu.VMEM((K, DH), _f32),
        pltpu.VMEM((K, DH), _f32),
        pltpu.VMEM((K,), _i32),
        pltpu.VMEM((K,), _i32),
        pltpu.VMEM((K,), _i32),
        pltpu.VMEM((K,), _f32),
        pltpu.VMEM((K,), _f32),
        pltpu.VMEM((K,), _f32),
        pltpu.SemaphoreType.DMA,
        pltpu.SemaphoreType.DMA,
        pltpu.SemaphoreType.DMA,
        pltpu.SemaphoreType.DMA,
        pltpu.SemaphoreType.DMA,
        pltpu.SemaphoreType.DMA,
        pltpu.SemaphoreType.DMA,
        pltpu.SemaphoreType.DMA,
        pltpu.SemaphoreType.DMA,
    ],
)(_propagation_kernel)

_score = functools.partial(
    pl.kernel, mesh=_mesh, compiler_params=_params,
    out_type=jax.ShapeDtypeStruct((BATCH,), _f32),
    scratch_types=[
        pltpu.VMEM((SB,), _i32),
        pltpu.VMEM((SB,), _i32),
        pltpu.VMEM((SB, DH), _f32),
        pltpu.VMEM((SB, DH), _f32),
        pltpu.VMEM((SB, DH), _f32),
        pltpu.VMEM((SB, DH), _f32),
        pltpu.VMEM((SB, DH), _f32),
        pltpu.VMEM((CB,), _f32),
        pltpu.SemaphoreType.DMA,
    ],
)(_score_kernel)


def kernel(user_id, item_id, user_table, item_table,
           ui_row, ui_col, ui_val, soc_row, soc_col, soc_val):
    pad = ((0, 0), (0, N_PAD - N_NODES), (0, 0))
    usr2 = jnp.pad(user_table.reshape(N_NODES, NC, DH).transpose(1, 0, 2),
                   pad)
    usr2 = usr2.reshape(NC * N_PAD, DH)
    itm2 = jnp.pad(item_table.reshape(N_NODES, NC, DH).transpose(1, 0, 2),
                   pad)
    itm2 = itm2.reshape(NC * N_PAD, DH)

    ui_row = ui_row.astype(_i32)
    ui_col = ui_col.astype(_i32)
    soc_row = soc_row.astype(_i32)
    soc_col = soc_col.astype(_i32)

    tbuf = _propagate(usr2, itm2,
                      ui_col, ui_row, soc_col,
                      ui_row, ui_col, soc_row,
                      ui_val * _f32(1.0 - SOCIAL_W),
                      ui_val,
                      soc_val * _f32(SOCIAL_W))
    t6 = tbuf.reshape(N_SLOTS, NC * N_PAD, DH)
    return _score(usr2, itm2, t6[0], t6[2], t6[4], t6[1], t6[3], t6[5],
                  user_id.astype(_i32), item_id.astype(_i32))
